# Initial kernel scaffold; baseline (speedup 1.0000x reference)
import functools

import jax
import jax.numpy as jnp
from jax import lax
from jax.experimental import pallas as pl
from jax.experimental.pallas import tpu as pltpu
from jax.experimental.pallas import tpu_sc as plsc

NUM_BINS = 100000
EMBED_DIM = 64
MIN_VAL = 0.0
MAX_VAL = 1.0
BIN_SIZE = (MAX_VAL - MIN_VAL) / (NUM_BINS - 1)

NC = 2
NS = 16
L = 16
NW = NC * NS

B, SEQ = 4096, 200
N = B * SEQ
PER_W = N // NW
CH = 128
NCH = PER_W // CH


def _sc_body(x_hbm, table_hbm, out_hbm, xv, ilo, ihi, dl, rlo, rhi, ob):
    wid = lax.axis_index("s") * NC + lax.axis_index("c")
    base = wid * PER_W
    pltpu.sync_copy(x_hbm.at[pl.ds(base, PER_W)], xv)

    @pl.loop(0, NCH)
    def _chunk(c):
        off = c * CH
        for g in range(CH // L):
            xg = xv[pl.ds(off + g * L, L)]
            xc = jnp.minimum(jnp.maximum(xg, MIN_VAL), MAX_VAL)
            ind = xc / jnp.float32(BIN_SIZE)
            lo = jnp.minimum(ind.astype(jnp.int32), NUM_BINS - 2)
            dl[pl.ds(g * L, L)] = ind - lo.astype(jnp.float32)
            ilo[pl.ds(g * L, L)] = lo
            ihi[pl.ds(g * L, L)] = lo + 1

        def _gathers(gsem):
            cp1 = pltpu.async_copy(table_hbm.at[ilo], rlo, gsem)
            cp2 = pltpu.async_copy(table_hbm.at[ihi], rhi, gsem)
            cp1.wait()
            cp2.wait()

        pl.run_scoped(_gathers, pltpu.SemaphoreType.DMA)

        @pl.loop(0, CH)
        def _elem(e):
            evec = lax.broadcast_in_dim(e, (L,), ())
            dsp = plsc.load_gather(dl, [evec])
            for q in range(EMBED_DIM // L):
                a = rlo[e, pl.ds(q * L, L)]
                b = rhi[e, pl.ds(q * L, L)]
                ob[e, pl.ds(q * L, L)] = a + dsp * (b - a)

        pltpu.sync_copy(ob, out_hbm.at[pl.ds(base + off, CH)])


@jax.jit
def kernel(x, table):
    xf = x.reshape(N)
    mesh = plsc.VectorSubcoreMesh(
        core_axis_name="c", subcore_axis_name="s", num_cores=NC, num_subcores=NS
    )
    call = pl.kernel(
        _sc_body,
        out_type=jax.ShapeDtypeStruct((N, EMBED_DIM), jnp.float32),
        mesh=mesh,
        scratch_types=[
            pltpu.VMEM((PER_W,), jnp.float32),
            pltpu.VMEM((CH,), jnp.int32),
            pltpu.VMEM((CH,), jnp.int32),
            pltpu.VMEM((CH,), jnp.float32),
            pltpu.VMEM((CH, EMBED_DIM), jnp.float32),
            pltpu.VMEM((CH, EMBED_DIM), jnp.float32),
            pltpu.VMEM((CH, EMBED_DIM), jnp.float32),
        ],
    )
    out = call(xf, table)
    return out.reshape(B, SEQ, EMBED_DIM)


# speedup vs baseline: 3.4261x; 3.4261x over previous
import functools

import jax
import jax.numpy as jnp
from jax import lax
from jax.experimental import pallas as pl
from jax.experimental.pallas import tpu as pltpu
from jax.experimental.pallas import tpu_sc as plsc

NUM_BINS = 100000
EMBED_DIM = 64
MIN_VAL = 0.0
MAX_VAL = 1.0
BIN_SIZE = (MAX_VAL - MIN_VAL) / (NUM_BINS - 1)

NC = 2
NS = 16
L = 16
NW = NC * NS

B, SEQ = 4096, 200
N = B * SEQ
PER_W = N // NW
CH = 128
NCH = PER_W // CH


def _sc_body(x_hbm, table_hbm, out_hbm, xv, ilo, ihi, dl, rlo, rhi, ob):
    wid = lax.axis_index("s") * NC + lax.axis_index("c")
    base = wid * PER_W
    pltpu.sync_copy(x_hbm.at[pl.ds(base, PER_W)], xv)

    @pl.loop(0, NCH)
    def _chunk(c):
        off = c * CH
        for g in range(CH // L):
            xg = xv[pl.ds(off + g * L, L)]
            xc = jnp.minimum(jnp.maximum(xg, MIN_VAL), MAX_VAL)
            ind = xc / jnp.float32(BIN_SIZE)
            lo = jnp.minimum(ind.astype(jnp.int32), NUM_BINS - 2)
            dl[pl.ds(g * L, L)] = ind - lo.astype(jnp.float32)
            ilo[pl.ds(g * L, L)] = lo
            ihi[pl.ds(g * L, L)] = lo + 1

        def _gathers(gsem):
            cp1 = pltpu.async_copy(table_hbm.at[ilo], rlo, gsem)
            cp2 = pltpu.async_copy(table_hbm.at[ihi], rhi, gsem)
            cp1.wait()
            cp2.wait()

        pl.run_scoped(_gathers, pltpu.SemaphoreType.DMA)

        @pl.loop(0, CH)
        def _elem(e):
            evec = lax.broadcast_in_dim(e, (L,), ())
            dsp = plsc.load_gather(dl, [evec])
            for q in range(EMBED_DIM // L):
                a = rlo[e, pl.ds(q * L, L)]
                b = rhi[e, pl.ds(q * L, L)]
                ob[e, pl.ds(q * L, L)] = a + dsp * (b - a)

        pltpu.sync_copy(ob, out_hbm.at[pl.ds(base + off, CH)])


@jax.jit
def kernel(x, table):
    xf = x.reshape(N)
    mesh = plsc.VectorSubcoreMesh(
        core_axis_name="c", subcore_axis_name="s", num_cores=NC, num_subcores=NS
    )
    call = pl.kernel(
        _sc_body,
        out_type=jax.ShapeDtypeStruct((N, EMBED_DIM), jnp.float32),
        mesh=mesh,
        compiler_params=pltpu.CompilerParams(
            needs_layout_passes=False, use_tc_tiling_on_sc=False
        ),
        scratch_types=[
            pltpu.VMEM((PER_W,), jnp.float32),
            pltpu.VMEM((CH,), jnp.int32),
            pltpu.VMEM((CH,), jnp.int32),
            pltpu.VMEM((CH,), jnp.float32),
            pltpu.VMEM((CH, EMBED_DIM), jnp.float32),
            pltpu.VMEM((CH, EMBED_DIM), jnp.float32),
            pltpu.VMEM((CH, EMBED_DIM), jnp.float32),
        ],
    )
    out = call(xf, table)
    return out.reshape(B, SEQ, EMBED_DIM)


# speedup vs baseline: 4.2984x; 1.2546x over previous
import functools

import jax
import jax.numpy as jnp
from jax import lax
from jax.experimental import pallas as pl
from jax.experimental.pallas import tpu as pltpu
from jax.experimental.pallas import tpu_sc as plsc

NUM_BINS = 100000
EMBED_DIM = 64
MIN_VAL = 0.0
MAX_VAL = 1.0
BIN_SIZE = (MAX_VAL - MIN_VAL) / (NUM_BINS - 1)

NC = 2
NS = 16
L = 16
NW = NC * NS

B, SEQ = 4096, 200
N = B * SEQ
PER_W = N // NW
CH = 128
NCH = PER_W // CH


def _sc_body(x_hbm, table_hbm, out_hbm, xv, *bufs):
    wid = lax.axis_index("s") * NC + lax.axis_index("c")
    base = wid * PER_W
    pltpu.sync_copy(x_hbm.at[pl.ds(base, PER_W)], xv)

    setA = bufs[0:8]
    setB = bufs[8:16]

    def prep_fire(c, S):
        ilo, ihi, dl, rlo, rhi, _ob, gsem, _osem = S
        off = c * CH
        for g in range(CH // L):
            xg = xv[pl.ds(off + g * L, L)]
            xc = jnp.minimum(jnp.maximum(xg, MIN_VAL), MAX_VAL)
            ind = xc / jnp.float32(BIN_SIZE)
            lo = jnp.minimum(ind.astype(jnp.int32), NUM_BINS - 2)
            dl[pl.ds(g * L, L)] = ind - lo.astype(jnp.float32)
            ilo[pl.ds(g * L, L)] = lo
            ihi[pl.ds(g * L, L)] = lo + 1
        pltpu.async_copy(table_hbm.at[ilo], rlo, gsem)
        pltpu.async_copy(table_hbm.at[ihi], rhi, gsem)

    def wait_g(S):
        ilo, ihi, _dl, rlo, rhi, _ob, gsem, _osem = S
        pltpu.make_async_copy(table_hbm.at[ilo], rlo, gsem).wait()
        pltpu.make_async_copy(table_hbm.at[ihi], rhi, gsem).wait()

    def wait_o(S):
        ob, osem = S[5], S[7]
        pltpu.make_async_copy(ob, out_hbm.at[pl.ds(0, CH)], osem).wait()

    def lerp_fire_out(c, S):
        _ilo, _ihi, dl, rlo, rhi, ob, _gsem, osem = S

        @pl.loop(0, CH, unroll=8)
        def _elem(e):
            evec = lax.broadcast_in_dim(e, (L,), ())
            dsp = plsc.load_gather(dl, [evec])
            for q in range(EMBED_DIM // L):
                a = rlo[e, pl.ds(q * L, L)]
                b = rhi[e, pl.ds(q * L, L)]
                ob[e, pl.ds(q * L, L)] = a + dsp * (b - a)

        pltpu.async_copy(ob, out_hbm.at[pl.ds(base + c * CH, CH)], osem)

    prep_fire(0, setA)
    prep_fire(1, setB)
    wait_g(setA)
    lerp_fire_out(0, setA)
    prep_fire(2, setA)
    wait_g(setB)
    lerp_fire_out(1, setB)
    prep_fire(3, setB)

    @pl.loop(0, (NCH - 4) // 2)
    def _pair(p):
        c0 = 2 * p + 2
        wait_g(setA)
        wait_o(setA)
        lerp_fire_out(c0, setA)
        prep_fire(c0 + 2, setA)
        wait_g(setB)
        wait_o(setB)
        lerp_fire_out(c0 + 1, setB)
        prep_fire(c0 + 3, setB)

    wait_g(setA)
    wait_o(setA)
    lerp_fire_out(NCH - 2, setA)
    wait_g(setB)
    wait_o(setB)
    lerp_fire_out(NCH - 1, setB)
    wait_o(setA)
    wait_o(setB)


def _buf_set():
    return [
        pltpu.VMEM((CH,), jnp.int32),
        pltpu.VMEM((CH,), jnp.int32),
        pltpu.VMEM((CH,), jnp.float32),
        pltpu.VMEM((CH, EMBED_DIM), jnp.float32),
        pltpu.VMEM((CH, EMBED_DIM), jnp.float32),
        pltpu.VMEM((CH, EMBED_DIM), jnp.float32),
        pltpu.SemaphoreType.DMA,
        pltpu.SemaphoreType.DMA,
    ]


@jax.jit
def kernel(x, table):
    xf = x.reshape(N)
    mesh = plsc.VectorSubcoreMesh(
        core_axis_name="c", subcore_axis_name="s", num_cores=NC, num_subcores=NS
    )
    call = pl.kernel(
        _sc_body,
        out_type=jax.ShapeDtypeStruct((N, EMBED_DIM), jnp.float32),
        mesh=mesh,
        compiler_params=pltpu.CompilerParams(
            needs_layout_passes=False, use_tc_tiling_on_sc=False
        ),
        scratch_types=[pltpu.VMEM((PER_W,), jnp.float32)] + _buf_set() + _buf_set(),
    )
    out = call(xf, table)
    return out.reshape(B, SEQ, EMBED_DIM)


# speedup vs baseline: 6.9310x; 1.6124x over previous
import functools

import jax
import jax.numpy as jnp
from jax import lax
from jax.experimental import pallas as pl
from jax.experimental.pallas import tpu as pltpu
from jax.experimental.pallas import tpu_sc as plsc

NUM_BINS = 100000
EMBED_DIM = 64
MIN_VAL = 0.0
MAX_VAL = 1.0
BIN_SIZE = (MAX_VAL - MIN_VAL) / (NUM_BINS - 1)

NC = 2
NS = 16
L = 16
NW = NC * NS

B, SEQ = 4096, 200
N = B * SEQ
PER_W = N // NW
CH = 128
NCH = PER_W // CH


def _sc_body(x_hbm, table_hbm, out_hbm, xv, *bufs):
    wid = lax.axis_index("s") * NC + lax.axis_index("c")
    base = wid * PER_W
    pltpu.sync_copy(x_hbm.at[pl.ds(base, PER_W)], xv)

    setA = bufs[0:8]
    setB = bufs[8:16]

    def prep_fire(c, S):
        ilo, ihi, dl, rlo, rhi, _ob, gsem, _osem = S
        off = c * CH
        for g in range(CH // L):
            xg = xv[pl.ds(off + g * L, L)]
            xc = jnp.minimum(jnp.maximum(xg, MIN_VAL), MAX_VAL)
            ind = xc / jnp.float32(BIN_SIZE)
            lo = jnp.minimum(ind.astype(jnp.int32), NUM_BINS - 2)
            dl[pl.ds(g * L, L)] = ind - lo.astype(jnp.float32)
            ilo[pl.ds(g * L, L)] = lo
            ihi[pl.ds(g * L, L)] = lo + 1
        pltpu.async_copy(table_hbm.at[ilo], rlo, gsem)
        pltpu.async_copy(table_hbm.at[ihi], rhi, gsem)

    def wait_g(S):
        ilo, ihi, _dl, rlo, rhi, _ob, gsem, _osem = S
        pltpu.make_async_copy(table_hbm.at[ilo], rlo, gsem).wait()
        pltpu.make_async_copy(table_hbm.at[ihi], rhi, gsem).wait()

    def wait_o(S):
        ob, osem = S[5], S[7]
        pltpu.make_async_copy(ob, out_hbm.at[pl.ds(0, CH)], osem).wait()

    Q = EMBED_DIM // L

    def lerp_fire_out(c, S):
        _ilo, _ihi, dl, rlo, rhi, ob, _gsem, osem = S

        @pl.loop(0, CH // 2, unroll=4)
        def _pair(i):
            e0 = 2 * i
            e1 = e0 + 1
            d0 = plsc.load_gather(dl, [lax.broadcast_in_dim(e0, (L,), ())])
            d1 = plsc.load_gather(dl, [lax.broadcast_in_dim(e1, (L,), ())])
            a = [rlo[e, pl.ds(q * L, L)] for e in (e0, e1) for q in range(Q)]
            b = [rhi[e, pl.ds(q * L, L)] for e in (e0, e1) for q in range(Q)]
            t = [bb - aa for aa, bb in zip(a, b)]
            m = [tt * (d0 if k < Q else d1) for k, tt in enumerate(t)]
            o = [aa + mm for aa, mm in zip(a, m)]
            for q in range(Q):
                ob[e0, pl.ds(q * L, L)] = o[q]
            for q in range(Q):
                ob[e1, pl.ds(q * L, L)] = o[Q + q]

        pltpu.async_copy(ob, out_hbm.at[pl.ds(base + c * CH, CH)], osem)

    prep_fire(0, setA)
    prep_fire(1, setB)
    wait_g(setA)
    lerp_fire_out(0, setA)
    prep_fire(2, setA)
    wait_g(setB)
    lerp_fire_out(1, setB)
    prep_fire(3, setB)

    @pl.loop(0, (NCH - 4) // 2)
    def _pair(p):
        c0 = 2 * p + 2
        wait_g(setA)
        wait_o(setA)
        lerp_fire_out(c0, setA)
        prep_fire(c0 + 2, setA)
        wait_g(setB)
        wait_o(setB)
        lerp_fire_out(c0 + 1, setB)
        prep_fire(c0 + 3, setB)

    wait_g(setA)
    wait_o(setA)
    lerp_fire_out(NCH - 2, setA)
    wait_g(setB)
    wait_o(setB)
    lerp_fire_out(NCH - 1, setB)
    wait_o(setA)
    wait_o(setB)


def _buf_set():
    return [
        pltpu.VMEM((CH,), jnp.int32),
        pltpu.VMEM((CH,), jnp.int32),
        pltpu.VMEM((CH,), jnp.float32),
        pltpu.VMEM((CH, EMBED_DIM), jnp.float32),
        pltpu.VMEM((CH, EMBED_DIM), jnp.float32),
        pltpu.VMEM((CH, EMBED_DIM), jnp.float32),
        pltpu.SemaphoreType.DMA,
        pltpu.SemaphoreType.DMA,
    ]


@jax.jit
def kernel(x, table):
    xf = x.reshape(N)
    mesh = plsc.VectorSubcoreMesh(
        core_axis_name="c", subcore_axis_name="s", num_cores=NC, num_subcores=NS
    )
    call = pl.kernel(
        _sc_body,
        out_type=jax.ShapeDtypeStruct((N, EMBED_DIM), jnp.float32),
        mesh=mesh,
        compiler_params=pltpu.CompilerParams(
            needs_layout_passes=False, use_tc_tiling_on_sc=False
        ),
        scratch_types=[pltpu.VMEM((PER_W,), jnp.float32)] + _buf_set() + _buf_set(),
    )
    out = call(xf, table)
    return out.reshape(B, SEQ, EMBED_DIM)
